# Initial kernel scaffold; baseline (speedup 1.0000x reference)
import jax
import jax.numpy as jnp
from jax.experimental import pallas as pl


def kernel(x_dealer, x_intent, x_fix, edge_has_intent, edge_resolved_by, edge_applies_fix, W_enc_d, b_enc_d, W_enc_i, b_enc_i, W_enc_f, b_enc_f, W1_hi, b1_hi, W1_rb, b1_rb, W1_af, b1_af, W2_hi, b2_hi, W2_rb, b2_rb, W2_af, b2_af, Wp1, bp1, Wp2, bp2, Wp3, bp3):
    raise NotImplementedError("write your pallas kernel here")


# speedup vs baseline: 2.1812x; 2.1812x over previous
import functools

import jax
import jax.numpy as jnp
from jax.experimental import pallas as pl
from jax.experimental.pallas import tpu as pltpu

ND, NI, NF = 100000, 2000, 64
H = 256
E_HI, E_RB, E_AF = 200000, 64000, 200000

_INTERPRET = False


def _dsi(deg):
    return jnp.where(deg > 0, jax.lax.rsqrt(jnp.maximum(deg, 1e-12)), 0.0)


def _enc_body(x_ref, w_ref, b_ref, hhi_ref, haf_ref, out_hi_ref, out_af_ref):
    d = jax.nn.relu(jnp.dot(x_ref[...], w_ref[...],
                            preferred_element_type=jnp.float32, precision=jax.lax.Precision.HIGHEST) + b_ref[...])
    dsi_hi = _dsi(hhi_ref[0, :] + hhi_ref[1, :])
    dsi_af = _dsi(haf_ref[0, :] + haf_ref[1, :])
    out_hi_ref[...] = d * dsi_hi[:, None]
    out_af_ref[...] = d * dsi_af[:, None]


def _encode_dealers(x_dealer, W, b, hs_hi, hs_af):
    TR = 2048
    grid = (pl.cdiv(ND, TR),)
    return pl.pallas_call(
        _enc_body,
        grid=grid,
        in_specs=[
            pl.BlockSpec((TR, 64), lambda i: (i, 0)),
            pl.BlockSpec((64, H), lambda i: (0, 0)),
            pl.BlockSpec((1, H), lambda i: (0, 0)),
            pl.BlockSpec((2, TR), lambda i: (0, i)),
            pl.BlockSpec((2, TR), lambda i: (0, i)),
        ],
        out_specs=[
            pl.BlockSpec((TR, H), lambda i: (i, 0)),
            pl.BlockSpec((TR, H), lambda i: (i, 0)),
        ],
        out_shape=[
            jax.ShapeDtypeStruct((ND, H), jnp.float32),
            jax.ShapeDtypeStruct((ND, H), jnp.float32),
        ],
        interpret=_INTERPRET,
    )(x_dealer, W, b.reshape(1, H), hs_hi, hs_af)


def _small_body(xi_ref, wei_ref, bei_ref,
                hdhi_ref, hsrb_ref, hdrb_ref, hdaf_ref, cnt_ref,
                agghi_ref, aggaf_ref,
                w1hi_ref, b1hi_ref, w2hi_ref, b2hi_ref,
                w2rb_ref, b2rb_ref, w2af_ref, b2af_ref,
                wp1a_ref, wp1b_ref, bp1_ref,
                a_ref, bmat_ref):
    f32 = jnp.float32
    ddi_hi = _dsi(hdhi_ref[0, :NI] + hdhi_ref[1, :NI])
    dsi_rb = _dsi(hsrb_ref[0, :NI] + hsrb_ref[1, :NI])
    ddi_rb = _dsi(hdrb_ref[0, :NF] + hdrb_ref[1, :NF])
    ddi_af = _dsi(hdaf_ref[0, :NF] + hdaf_ref[1, :NF])

    agg_hi = agghi_ref[0] + agghi_ref[1]
    agg_af = aggaf_ref[0] + aggaf_ref[1]

    ii = jax.nn.relu(jnp.dot(xi_ref[...], wei_ref[...],
                             preferred_element_type=f32, precision=jax.lax.Precision.HIGHEST) + bei_ref[...])
    M = (cnt_ref[0] + cnt_ref[1]) * dsi_rb[None, :]

    i1 = ddi_hi[:, None] * jnp.dot(agg_hi, w1hi_ref[...],
                                   preferred_element_type=f32, precision=jax.lax.Precision.HIGHEST) + b1hi_ref[...]
    i2 = ddi_hi[:, None] * jnp.dot(agg_hi, w2hi_ref[...],
                                   preferred_element_type=f32, precision=jax.lax.Precision.HIGHEST) + b2hi_ref[...]
    i1r = jax.nn.relu(i1)
    g2 = jnp.dot(M, i1r, preferred_element_type=f32, precision=jax.lax.Precision.HIGHEST)
    f2 = (ddi_rb[:, None] * jnp.dot(g2, w2rb_ref[...], preferred_element_type=f32, precision=jax.lax.Precision.HIGHEST)
          + b2rb_ref[...]
          + ddi_af[:, None] * jnp.dot(agg_af, w2af_ref[...],
                                      preferred_element_type=f32, precision=jax.lax.Precision.HIGHEST) + b2af_ref[...])

    a_ref[...] = jnp.dot(i2, wp1a_ref[...], preferred_element_type=f32, precision=jax.lax.Precision.HIGHEST)
    bmat_ref[...] = jnp.dot(f2, wp1b_ref[...], preferred_element_type=f32, precision=jax.lax.Precision.HIGHEST) + bp1_ref[...]


def _small_stage(x_intent, W_enc_i, b_enc_i, hd_hi, hs_rb, hd_rb, hd_af, cnt,
                 agg_hi_p, agg_af_p, W1_hi, b1_hi, W2_hi, b2_hi,
                 W2_rb, b2_rb, W2_af, b2_af, Wp1, bp1):
    return pl.pallas_call(
        _small_body,
        out_shape=[
            jax.ShapeDtypeStruct((NI, H), jnp.float32),
            jax.ShapeDtypeStruct((NF, H), jnp.float32),
        ],
        interpret=_INTERPRET,
    )(x_intent, W_enc_i, b_enc_i.reshape(1, H),
      hd_hi, hs_rb, hd_rb, hd_af, cnt,
      agg_hi_p, agg_af_p,
      W1_hi, b1_hi.reshape(1, H), W2_hi, b2_hi.reshape(1, H),
      W2_rb, b2_rb.reshape(1, H), W2_af, b2_af.reshape(1, H),
      Wp1[:H], Wp1[H:], bp1.reshape(1, H))


def _pair_body(a_ref, b_ref, wp2_ref, bp2_ref, wp3_ref, bp3_ref, out_ref, *, tr):
    f32 = jnp.float32
    h1 = jax.nn.relu(a_ref[...][:, None, :] + b_ref[...][None, :, :])
    h1 = h1.reshape(tr * NF, H)
    h2 = jax.nn.relu(jnp.dot(h1, wp2_ref[...], preferred_element_type=f32, precision=jax.lax.Precision.HIGHEST)
                     + bp2_ref[...])
    logit = jnp.dot(h2, wp3_ref[...], preferred_element_type=f32, precision=jax.lax.Precision.HIGHEST) + bp3_ref[...]
    out_ref[...] = jax.nn.sigmoid(logit.reshape(tr, NF))


def _pairwise(A, B, Wp2, bp2, Wp3, bp3):
    TR = 200
    grid = (NI // TR,)
    return pl.pallas_call(
        functools.partial(_pair_body, tr=TR),
        grid=grid,
        in_specs=[
            pl.BlockSpec((TR, H), lambda i: (i, 0)),
            pl.BlockSpec((NF, H), lambda i: (0, 0)),
            pl.BlockSpec((H, H // 2), lambda i: (0, 0)),
            pl.BlockSpec((1, H // 2), lambda i: (0, 0)),
            pl.BlockSpec((H // 2, 1), lambda i: (0, 0)),
            pl.BlockSpec((1, 1), lambda i: (0, 0)),
        ],
        out_specs=pl.BlockSpec((TR, NF), lambda i: (i, 0)),
        out_shape=jax.ShapeDtypeStruct((NI, NF), jnp.float32),
        interpret=_INTERPRET,
    )(A, B, Wp2, bp2.reshape(1, H // 2), Wp3, bp3.reshape(1, 1))


def kernel(x_dealer, x_intent, x_fix, edge_has_intent, edge_resolved_by,
           edge_applies_fix, W_enc_d, b_enc_d, W_enc_i, b_enc_i, W_enc_f,
           b_enc_f, W1_hi, b1_hi, W1_rb, b1_rb, W1_af, b1_af, W2_hi, b2_hi,
           W2_rb, b2_rb, W2_af, b2_af, Wp1, bp1, Wp2, bp2, Wp3, bp3):
    ehi, erb, eaf = edge_has_intent, edge_resolved_by, edge_applies_fix

    def hist(idx, n):
        return jnp.zeros((n,), jnp.float32).at[idx].add(1.0)

    NPD = 100352
    hs_hi = jnp.stack([hist(ehi[0], ND), jnp.zeros((ND,), jnp.float32)])
    hs_af = jnp.stack([hist(eaf[0], ND), jnp.zeros((ND,), jnp.float32)])
    hs_hi = jnp.pad(hs_hi, ((0, 0), (0, NPD - ND)))
    hs_af = jnp.pad(hs_af, ((0, 0), (0, NPD - ND)))
    hd_hi = jnp.stack([hist(ehi[1], NI), jnp.zeros((NI,), jnp.float32)])
    hs_rb = jnp.stack([hist(erb[0], NI), jnp.zeros((NI,), jnp.float32)])
    hd_rb = jnp.stack([hist(erb[1], NF), jnp.zeros((NF,), jnp.float32)])
    hd_af = jnp.stack([hist(eaf[1], NF), jnp.zeros((NF,), jnp.float32)])
    cnt = jnp.zeros((NF * NI,), jnp.float32).at[erb[1] * NI + erb[0]].add(1.0)
    cnt = jnp.stack([cnt.reshape(NF, NI), jnp.zeros((NF, NI), jnp.float32)])

    d_hi, d_af = _encode_dealers(x_dealer, W_enc_d, b_enc_d, hs_hi, hs_af)

    agg_hi = jnp.zeros((NI, H), jnp.float32).at[ehi[1]].add(d_hi[ehi[0]])
    agg_af = jnp.zeros((NF, H), jnp.float32).at[eaf[1]].add(d_af[eaf[0]])
    agg_hi_p = jnp.stack([agg_hi, jnp.zeros((NI, H), jnp.float32)])
    agg_af_p = jnp.stack([agg_af, jnp.zeros((NF, H), jnp.float32)])

    A, B = _small_stage(x_intent, W_enc_i, b_enc_i, hd_hi, hs_rb, hd_rb, hd_af,
                        cnt, agg_hi_p, agg_af_p, W1_hi, b1_hi, W2_hi, b2_hi,
                        W2_rb, b2_rb, W2_af, b2_af, Wp1, bp1)
    return _pairwise(A, B, Wp2, bp2, Wp3, bp3)


# speedup vs baseline: 2.4038x; 1.1020x over previous
import functools

import jax
import jax.numpy as jnp
from jax import lax
from jax.experimental import pallas as pl
from jax.experimental.pallas import tpu as pltpu
from jax.experimental.pallas import tpu_sc as plsc

ND, NI, NF = 100000, 2000, 64
H = 256
E_HI, E_RB, E_AF = 200000, 64000, 200000

NPD = 100352
T_SM, T_F = 2048, 256
T_CNT_H = 65024
CH = 2048
PAD_BIG = 200704
PAD_RB = 65536
PAD_Q = 204800
NCH_Q = 25
GF = 32
NG = 8
THI_ROWS = 2048
TAF_ROWS = 128

_INTERPRET = False


def _pad1(a, tot, fill):
    return jnp.pad(a, (0, tot - a.shape[0]), constant_values=jnp.int32(fill))


def _sc_hist(h_ehi0, h_ehi1, h_eaf0, h_eaf1, h_erb0, h_erb1, h_flat):
    f32 = jnp.float32
    i32 = jnp.int32
    mesh = plsc.VectorSubcoreMesh(core_axis_name="c", subcore_axis_name="s")

    @functools.partial(
        pl.kernel,
        out_type=[
            jax.ShapeDtypeStruct((32 * NPD,), f32),
            jax.ShapeDtypeStruct((32 * T_SM,), f32),
            jax.ShapeDtypeStruct((32 * NPD,), f32),
            jax.ShapeDtypeStruct((32 * T_F,), f32),
            jax.ShapeDtypeStruct((32 * T_SM,), f32),
            jax.ShapeDtypeStruct((32 * T_F,), f32),
            jax.ShapeDtypeStruct((32 * 2 * T_CNT_H,), f32),
        ],
        mesh=mesh,
        scratch_types=[
            pltpu.VMEM((NPD,), f32),
            pltpu.VMEM((PAD_BIG // 32,), i32),
        ],
        compiler_params=pltpu.CompilerParams(needs_layout_passes=False),
    )
    def k(e0_h, e1_h, a0_h, a1_h, r0_h, r1_h, fl_h,
          o0, o1, o2, o3, o4, o5, o6,
          tb, eb):
        c = lax.axis_index("c")
        s = lax.axis_index("s")
        u = c * 16 + s
        iota = lax.iota(i32, 16)
        lmask = [iota == l for l in range(16)]
        ones16 = jnp.ones((16,), f32)

        phases = (
            (e0_h, PAD_BIG, NPD, o0, NPD, 0, False),
            (e1_h, PAD_BIG, T_SM, o1, T_SM, 0, False),
            (a0_h, PAD_BIG, NPD, o2, NPD, 0, False),
            (a1_h, PAD_BIG, T_F, o3, T_F, 0, False),
            (r0_h, PAD_RB, T_SM, o4, T_SM, 0, False),
            (r1_h, PAD_RB, T_F, o5, T_F, 0, False),
            (fl_h, PAD_RB, T_CNT_H, o6, 2 * T_CNT_H, 0, True),
            (fl_h, PAD_RB, T_CNT_H, o6, 2 * T_CNT_H, T_CNT_H, True),
        )
        for arr_h, epad, nbins, out, ostride, binlo, ranged in phases:
            epp = epad // 32

            def zt(j, _):
                tb[pl.ds(j * 16, 16)] = jnp.zeros((16,), f32)
                return _
            lax.fori_loop(0, nbins // 16, zt, None)

            pltpu.sync_copy(arr_h.at[pl.ds(u * epp, epp)], eb.at[pl.ds(0, epp)])

            def sv(j, _, nbins=nbins, binlo=binlo, ranged=ranged):
                v = eb[pl.ds(j * 16, 16)] - binlo
                if ranged:
                    m = (v >= 0) & (v < nbins)
                    for l in range(16):
                        plsc.addupdate_scatter(tb, [v], ones16,
                                               mask=m & lmask[l])
                else:
                    for l in range(16):
                        plsc.addupdate_scatter(tb, [v], ones16, mask=lmask[l])
                return _
            lax.fori_loop(0, epp // 16, sv, None)

            off = pl.multiple_of(u * ostride + binlo, 8)
            pltpu.sync_copy(tb.at[pl.ds(0, nbins)], out.at[pl.ds(off, nbins)])

    outs = k(h_ehi0, h_ehi1, h_eaf0, h_eaf1, h_erb0, h_erb1, h_flat)
    return [o.reshape(32, -1) for o in outs]


def _sc_agg(dhi_tabs, daf_tabs, hi_src, hi_dst, af_src, af_dst):
    f32 = jnp.float32
    i32 = jnp.int32
    mesh = plsc.VectorSubcoreMesh(core_axis_name="c", subcore_axis_name="s")

    @functools.partial(
        pl.kernel,
        out_type=[
            jax.ShapeDtypeStruct((32 * THI_ROWS * GF,), f32),
            jax.ShapeDtypeStruct((32 * TAF_ROWS * GF,), f32),
        ],
        mesh=mesh,
        scratch_types=[
            pltpu.VMEM((THI_ROWS * GF,), f32),
            pltpu.VMEM((TAF_ROWS * GF,), f32),
            pltpu.VMEM((CH,), i32),
            pltpu.VMEM((CH,), i32),
            pltpu.VMEM((128, GF), f32),
            pltpu.VMEM((128,), i32),
            pltpu.SemaphoreType.DMA,
        ],
        compiler_params=pltpu.CompilerParams(needs_layout_passes=False,
                                             use_tc_tiling_on_sc=False),
    )
    def k(dh0, dh1, dh2, dh3, dh4, dh5, dh6, dh7,
          da0, da1, da2, da3, da4, da5, da6, da7,
          hs_h, hd_h, as_h, ad_h,
          o_hi, o_af,
          t_hi, t_af, sbuf, dbuf, rows, idxs, sem):
        c = lax.axis_index("c")
        s = lax.axis_index("s")
        u = c * 16 + s
        q = u // NG
        g = u % NG
        iota = lax.iota(i32, 16)
        cols = [f * 16 + iota for f in range(GF // 16)]
        dhi = (dh0, dh1, dh2, dh3, dh4, dh5, dh6, dh7)
        daf = (da0, da1, da2, da3, da4, da5, da6, da7)

        for t, n in ((t_hi, THI_ROWS * GF), (t_af, TAF_ROWS * GF)):
            def zt(j, _, t=t):
                t[pl.ds(j * 16, 16)] = jnp.zeros((16,), f32)
                return _
            lax.fori_loop(0, n // 16, zt, None)

        for src_h, dst_h, tabs, t in ((hs_h, hd_h, dhi, t_hi),
                                      (as_h, ad_h, daf, t_af)):
            def chunk(kk, _, src_h=src_h, dst_h=dst_h, tabs=tabs, t=t):
                base = pl.multiple_of(q * (NCH_Q * CH) + kk * CH, 8)
                pltpu.sync_copy(src_h.at[pl.ds(base, CH)], sbuf)
                pltpu.sync_copy(dst_h.at[pl.ds(base, CH)], dbuf)

                def bloop(bb, _, tabs=tabs, t=t):
                    for qq in range(8):
                        idxs[pl.ds(qq * 16, 16)] = sbuf[pl.ds(bb * 128 + qq * 16, 16)]
                    for gi in range(NG):
                        @pl.when(g == gi)
                        def _gather(gi=gi, tabs=tabs):
                            pltpu.async_copy(tabs[gi].at[idxs], rows, sem).wait()

                    def grp(gr, _, t=t):
                        dlv = dbuf[pl.ds(bb * 128 + gr * 16, 16)] * GF
                        for l in range(16):
                            dl = jnp.sum(jnp.where(iota == l, dlv, jnp.int32(0)))
                            erow = jnp.broadcast_to(gr * 16 + l, (16,)).astype(i32)
                            for f in range(GF // 16):
                                val = plsc.load_gather(rows, [erow, cols[f]])
                                plsc.addupdate_scatter(t, [dl + cols[f]], val)
                        return _
                    lax.fori_loop(0, 8, grp, None)
                    return _
                lax.fori_loop(0, CH // 128, bloop, None)
                return _
            lax.fori_loop(0, NCH_Q, chunk, None)

        off_hi = pl.multiple_of(u * (THI_ROWS * GF), 8)
        pltpu.sync_copy(t_hi.at[pl.ds(0, THI_ROWS * GF)],
                        o_hi.at[pl.ds(off_hi, THI_ROWS * GF)])
        off_af = pl.multiple_of(u * (TAF_ROWS * GF), 8)
        pltpu.sync_copy(t_af.at[pl.ds(0, TAF_ROWS * GF)],
                        o_af.at[pl.ds(off_af, TAF_ROWS * GF)])

    o_hi, o_af = k(*dhi_tabs, *daf_tabs, hi_src, hi_dst, af_src, af_dst)
    o_hi = o_hi.reshape(4, NG, THI_ROWS, GF).transpose(0, 2, 1, 3)
    o_af = o_af.reshape(4, NG, TAF_ROWS, GF).transpose(0, 2, 1, 3)
    return (o_hi.reshape(4, THI_ROWS, H)[:, :NI],
            o_af.reshape(4, TAF_ROWS, H)[:, :NF])


def _dsi(deg):
    return jnp.where(deg > 0, jax.lax.rsqrt(jnp.maximum(deg, 1e-12)), 0.0)


def _enc_body(x_ref, w_ref, b_ref, hhi_ref, haf_ref, *out_refs):
    d = jax.nn.relu(jnp.dot(x_ref[...], w_ref[...],
                            preferred_element_type=jnp.float32,
                            precision=jax.lax.Precision.HIGHEST) + b_ref[...])
    dsi_hi = _dsi(jnp.sum(hhi_ref[...], axis=0))
    dsi_af = _dsi(jnp.sum(haf_ref[...], axis=0))
    d_hi = d * dsi_hi[:, None]
    d_af = d * dsi_af[:, None]
    for g in range(NG):
        out_refs[g][...] = d_hi[:, g * GF:(g + 1) * GF]
        out_refs[NG + g][...] = d_af[:, g * GF:(g + 1) * GF]


def _encode_dealers(x_dealer, W, b, hs_hi, hs_af):
    TR = 2048
    grid = (pl.cdiv(ND, TR),)
    return pl.pallas_call(
        _enc_body,
        grid=grid,
        in_specs=[
            pl.BlockSpec((TR, 64), lambda i: (i, 0)),
            pl.BlockSpec((64, H), lambda i: (0, 0)),
            pl.BlockSpec((1, H), lambda i: (0, 0)),
            pl.BlockSpec((32, TR), lambda i: (0, i)),
            pl.BlockSpec((32, TR), lambda i: (0, i)),
        ],
        out_specs=[pl.BlockSpec((TR, GF), lambda i: (i, 0))] * (2 * NG),
        out_shape=[jax.ShapeDtypeStruct((ND, GF), jnp.float32)] * (2 * NG),
        interpret=_INTERPRET,
    )(x_dealer, W, b.reshape(1, H), hs_hi, hs_af)


def _small_body(xi_ref, wei_ref, bei_ref,
                hdhi_ref, hsrb_ref, hdrb_ref, hdaf_ref, cnt_ref,
                agghi_ref, aggaf_ref,
                w1hi_ref, b1hi_ref, w2hi_ref, b2hi_ref,
                w2rb_ref, b2rb_ref, w2af_ref, b2af_ref,
                wp1a_ref, wp1b_ref, bp1_ref,
                a_ref, bmat_ref):
    f32 = jnp.float32
    hp = jax.lax.Precision.HIGHEST
    ddi_hi = _dsi(jnp.sum(hdhi_ref[...], axis=0)[:NI])
    dsi_rb = _dsi(jnp.sum(hsrb_ref[...], axis=0)[:NI])
    ddi_rb = _dsi(jnp.sum(hdrb_ref[...], axis=0)[:NF])
    ddi_af = _dsi(jnp.sum(hdaf_ref[...], axis=0)[:NF])

    agg_hi = jnp.sum(agghi_ref[...], axis=0)
    agg_af = jnp.sum(aggaf_ref[...], axis=0)

    ii = jax.nn.relu(jnp.dot(xi_ref[...], wei_ref[...],
                             preferred_element_type=f32, precision=hp)
                     + bei_ref[...])
    M = jnp.sum(cnt_ref[...], axis=0) * dsi_rb[None, :]

    i1 = ddi_hi[:, None] * jnp.dot(agg_hi, w1hi_ref[...],
                                   preferred_element_type=f32, precision=hp) \
        + b1hi_ref[...]
    i2 = ddi_hi[:, None] * jnp.dot(agg_hi, w2hi_ref[...],
                                   preferred_element_type=f32, precision=hp) \
        + b2hi_ref[...]
    i1r = jax.nn.relu(i1)
    g2 = jnp.dot(M, i1r, preferred_element_type=f32, precision=hp)
    f2 = (ddi_rb[:, None] * jnp.dot(g2, w2rb_ref[...],
                                    preferred_element_type=f32, precision=hp)
          + b2rb_ref[...]
          + ddi_af[:, None] * jnp.dot(agg_af, w2af_ref[...],
                                      preferred_element_type=f32, precision=hp)
          + b2af_ref[...])

    a_ref[...] = jnp.dot(i2, wp1a_ref[...], preferred_element_type=f32,
                         precision=hp)
    bmat_ref[...] = jnp.dot(f2, wp1b_ref[...], preferred_element_type=f32,
                            precision=hp) + bp1_ref[...]


def _small_stage(x_intent, W_enc_i, b_enc_i, hd_hi, hs_rb, hd_rb, hd_af, cnt,
                 agg_hi_p, agg_af_p, W1_hi, b1_hi, W2_hi, b2_hi,
                 W2_rb, b2_rb, W2_af, b2_af, Wp1, bp1):
    return pl.pallas_call(
        _small_body,
        out_shape=[
            jax.ShapeDtypeStruct((NI, H), jnp.float32),
            jax.ShapeDtypeStruct((NF, H), jnp.float32),
        ],
        interpret=_INTERPRET,
    )(x_intent, W_enc_i, b_enc_i.reshape(1, H),
      hd_hi, hs_rb, hd_rb, hd_af, cnt,
      agg_hi_p, agg_af_p,
      W1_hi, b1_hi.reshape(1, H), W2_hi, b2_hi.reshape(1, H),
      W2_rb, b2_rb.reshape(1, H), W2_af, b2_af.reshape(1, H),
      Wp1[:H], Wp1[H:], bp1.reshape(1, H))


def _pair_body(a_ref, b_ref, wp2_ref, bp2_ref, wp3_ref, bp3_ref, out_ref, *, tr):
    f32 = jnp.float32
    hp = jax.lax.Precision.HIGHEST
    h1 = jax.nn.relu(a_ref[...][:, None, :] + b_ref[...][None, :, :])
    h1 = h1.reshape(tr * NF, H)
    h2 = jax.nn.relu(jnp.dot(h1, wp2_ref[...], preferred_element_type=f32,
                             precision=hp) + bp2_ref[...])
    logit = jnp.dot(h2, wp3_ref[...], preferred_element_type=f32,
                    precision=hp) + bp3_ref[...]
    out_ref[...] = jax.nn.sigmoid(logit.reshape(tr, NF))


def _pairwise(A, B, Wp2, bp2, Wp3, bp3):
    TR = 200
    grid = (NI // TR,)
    return pl.pallas_call(
        functools.partial(_pair_body, tr=TR),
        grid=grid,
        in_specs=[
            pl.BlockSpec((TR, H), lambda i: (i, 0)),
            pl.BlockSpec((NF, H), lambda i: (0, 0)),
            pl.BlockSpec((H, H // 2), lambda i: (0, 0)),
            pl.BlockSpec((1, H // 2), lambda i: (0, 0)),
            pl.BlockSpec((H // 2, 1), lambda i: (0, 0)),
            pl.BlockSpec((1, 1), lambda i: (0, 0)),
        ],
        out_specs=pl.BlockSpec((TR, NF), lambda i: (i, 0)),
        out_shape=jax.ShapeDtypeStruct((NI, NF), jnp.float32),
        interpret=_INTERPRET,
    )(A, B, Wp2, bp2.reshape(1, H // 2), Wp3, bp3.reshape(1, 1))


def kernel(x_dealer, x_intent, x_fix, edge_has_intent, edge_resolved_by,
           edge_applies_fix, W_enc_d, b_enc_d, W_enc_i, b_enc_i, W_enc_f,
           b_enc_f, W1_hi, b1_hi, W1_rb, b1_rb, W1_af, b1_af, W2_hi, b2_hi,
           W2_rb, b2_rb, W2_af, b2_af, Wp1, bp1, Wp2, bp2, Wp3, bp3):
    ehi, erb, eaf = edge_has_intent, edge_resolved_by, edge_applies_fix

    flat_rb = erb[1] * NI + erb[0]
    hs_hi, hd_hi, hs_af, hd_af, hs_rb, hd_rb, cnt = _sc_hist(
        _pad1(ehi[0], PAD_BIG, ND), _pad1(ehi[1], PAD_BIG, NI),
        _pad1(eaf[0], PAD_BIG, ND), _pad1(eaf[1], PAD_BIG, NF),
        _pad1(erb[0], PAD_RB, NI), _pad1(erb[1], PAD_RB, NF),
        _pad1(flat_rb, PAD_RB, NF * NI + NI))
    cnt = cnt[:, :NF * NI].reshape(32, NF, NI)

    d_tabs = _encode_dealers(x_dealer, W_enc_d, b_enc_d, hs_hi, hs_af)

    agg_hi_p, agg_af_p = _sc_agg(
        d_tabs[:NG], d_tabs[NG:],
        _pad1(ehi[0], PAD_Q, 0), _pad1(ehi[1], PAD_Q, THI_ROWS - 1),
        _pad1(eaf[0], PAD_Q, 0), _pad1(eaf[1], PAD_Q, TAF_ROWS - 1))

    A, B = _small_stage(x_intent, W_enc_i, b_enc_i, hd_hi, hs_rb, hd_rb, hd_af,
                        cnt, agg_hi_p, agg_af_p, W1_hi, b1_hi, W2_hi, b2_hi,
                        W2_rb, b2_rb, W2_af, b2_af, Wp1, bp1)
    return _pairwise(A, B, Wp2, bp2, Wp3, bp3)


# speedup vs baseline: 3.1998x; 1.3312x over previous
import functools

import jax
import jax.numpy as jnp
from jax import lax
from jax.experimental import pallas as pl
from jax.experimental.pallas import tpu as pltpu
from jax.experimental.pallas import tpu_sc as plsc

ND, NI, NF = 100000, 2000, 64
H = 256
E_HI, E_RB, E_AF = 200000, 64000, 200000

NPD = 100352
T_SM, T_F = 2048, 256
T_CNT_H = 65024
CH = 2048
PAD_BIG = 200704
PAD_RB = 65536
PAD_Q = 204800
NCH_Q = 25
GF = 32
NG = 8
THI_ROWS = 2048
TAF_ROWS = 128

_INTERPRET = False


def _pad1(a, tot, fill):
    return jnp.pad(a, (0, tot - a.shape[0]), constant_values=jnp.int32(fill))


def _sc_hist(h_ehi0, h_ehi1, h_eaf0, h_eaf1, h_erb0, h_erb1, h_flat):
    f32 = jnp.float32
    i32 = jnp.int32
    mesh = plsc.VectorSubcoreMesh(core_axis_name="c", subcore_axis_name="s")

    @functools.partial(
        pl.kernel,
        out_type=[
            jax.ShapeDtypeStruct((32 * NPD,), f32),
            jax.ShapeDtypeStruct((32 * T_SM,), f32),
            jax.ShapeDtypeStruct((32 * NPD,), f32),
            jax.ShapeDtypeStruct((32 * T_F,), f32),
            jax.ShapeDtypeStruct((32 * T_SM,), f32),
            jax.ShapeDtypeStruct((32 * T_F,), f32),
            jax.ShapeDtypeStruct((32 * 2 * T_CNT_H,), f32),
        ],
        mesh=mesh,
        scratch_types=[
            pltpu.VMEM((NPD,), f32),
            pltpu.VMEM((PAD_BIG // 32,), i32),
        ],
        compiler_params=pltpu.CompilerParams(needs_layout_passes=False),
    )
    def k(e0_h, e1_h, a0_h, a1_h, r0_h, r1_h, fl_h,
          o0, o1, o2, o3, o4, o5, o6,
          tb, eb):
        c = lax.axis_index("c")
        s = lax.axis_index("s")
        u = c * 16 + s
        iota = lax.iota(i32, 16)
        lmask = [iota == l for l in range(16)]
        ones16 = jnp.ones((16,), f32)

        phases = (
            (e0_h, PAD_BIG, NPD, o0, NPD, 0, False),
            (e1_h, PAD_BIG, T_SM, o1, T_SM, 0, False),
            (a0_h, PAD_BIG, NPD, o2, NPD, 0, False),
            (a1_h, PAD_BIG, T_F, o3, T_F, 0, False),
            (r0_h, PAD_RB, T_SM, o4, T_SM, 0, False),
            (r1_h, PAD_RB, T_F, o5, T_F, 0, False),
            (fl_h, PAD_RB, T_CNT_H, o6, 2 * T_CNT_H, 0, True),
            (fl_h, PAD_RB, T_CNT_H, o6, 2 * T_CNT_H, T_CNT_H, True),
        )
        for arr_h, epad, nbins, out, ostride, binlo, ranged in phases:
            epp = epad // 32

            def zt(j, _):
                tb[pl.ds(j * 16, 16)] = jnp.zeros((16,), f32)
                return _
            lax.fori_loop(0, nbins // 16, zt, None)

            pltpu.sync_copy(arr_h.at[pl.ds(u * epp, epp)], eb.at[pl.ds(0, epp)])

            def sv(j, _, nbins=nbins, binlo=binlo, ranged=ranged):
                v = eb[pl.ds(j * 16, 16)] - binlo
                if ranged:
                    m = (v >= 0) & (v < nbins)
                    for l in range(16):
                        plsc.addupdate_scatter(tb, [v], ones16,
                                               mask=m & lmask[l])
                else:
                    for l in range(16):
                        plsc.addupdate_scatter(tb, [v], ones16, mask=lmask[l])
                return _
            lax.fori_loop(0, epp // 16, sv, None)

            off = pl.multiple_of(u * ostride + binlo, 8)
            pltpu.sync_copy(tb.at[pl.ds(0, nbins)], out.at[pl.ds(off, nbins)])

    outs = k(h_ehi0, h_ehi1, h_eaf0, h_eaf1, h_erb0, h_erb1, h_flat)
    return [o.reshape(32, -1) for o in outs]


def _sc_agg(d_hi_r, d_af_r, hi_src, hi_dst, af_src, af_dst):
    f32 = jnp.float32
    i32 = jnp.int32
    mesh = plsc.VectorSubcoreMesh(core_axis_name="c", subcore_axis_name="s")

    @functools.partial(
        pl.kernel,
        out_type=[
            jax.ShapeDtypeStruct((32 * THI_ROWS * GF,), f32),
            jax.ShapeDtypeStruct((32 * TAF_ROWS * GF,), f32),
        ],
        mesh=mesh,
        scratch_types=[
            pltpu.VMEM((THI_ROWS * GF,), f32),
            pltpu.VMEM((TAF_ROWS * GF,), f32),
            pltpu.VMEM((CH,), i32),
            pltpu.VMEM((CH,), i32),
            pltpu.VMEM((128, GF), f32),
            pltpu.VMEM((128, GF), f32),
            pltpu.VMEM((128,), i32),
            pltpu.VMEM((128,), i32),
            pltpu.SemaphoreType.DMA,
            pltpu.SemaphoreType.DMA,
        ],
        compiler_params=pltpu.CompilerParams(needs_layout_passes=False,
                                             use_tc_tiling_on_sc=False),
    )
    def k(dh_h, da_h, hs_h, hd_h, as_h, ad_h,
          o_hi, o_af,
          t_hi, t_af, sbuf, dbuf, rows0, rows1, idxs0, idxs1, sem0, sem1):
        c = lax.axis_index("c")
        s = lax.axis_index("s")
        u = c * 16 + s
        q = u // NG
        g = u % NG
        iota = lax.iota(i32, 16)
        cols = [f * 16 + iota for f in range(GF // 16)]
        rows_b = (rows0, rows1)
        idxs_b = (idxs0, idxs1)
        sems = (sem0, sem1)

        for t, n in ((t_hi, THI_ROWS * GF), (t_af, TAF_ROWS * GF)):
            def zt(j, _, t=t):
                t[pl.ds(j * 16, 16)] = jnp.zeros((16,), f32)
                return _
            lax.fori_loop(0, n // 16, zt, None)

        for src_h, dst_h, d_h, t in ((hs_h, hd_h, dh_h, t_hi),
                                     (as_h, ad_h, da_h, t_af)):
            def chunk(kk, _, src_h=src_h, dst_h=dst_h, d_h=d_h, t=t):
                base = pl.multiple_of(q * (NCH_Q * CH) + kk * CH, 8)
                pltpu.sync_copy(src_h.at[pl.ds(base, CH)], sbuf)
                pltpu.sync_copy(dst_h.at[pl.ds(base, CH)], dbuf)

                def xf(j, _):
                    sbuf[pl.ds(j * 16, 16)] = sbuf[pl.ds(j * 16, 16)] * NG + g
                    dbuf[pl.ds(j * 16, 16)] = dbuf[pl.ds(j * 16, 16)] * GF
                    return _
                lax.fori_loop(0, CH // 16, xf, None)

                def stage(bb, buf):
                    for qq in range(8):
                        idxs_b[buf][pl.ds(qq * 16, 16)] = \
                            sbuf[pl.ds(bb * 128 + qq * 16, 16)]
                    return pltpu.async_copy(d_h.at[idxs_b[buf]], rows_b[buf],
                                            sems[buf])

                desc = stage(0, 0)
                for bb in range(CH // 128):
                    cur = bb % 2
                    desc.wait()
                    if bb + 1 < CH // 128:
                        desc = stage(bb + 1, 1 - cur)

                    def grp(gr, _, t=t, cur=cur, bb=bb):
                        for l in range(16):
                            posv = jnp.broadcast_to(bb * 128 + gr * 16 + l,
                                                    (16,)).astype(i32)
                            dlb = plsc.load_gather(dbuf, [posv])
                            erow = posv - bb * 128
                            for f in range(GF // 16):
                                val = plsc.load_gather(rows_b[cur],
                                                       [erow, cols[f]])
                                plsc.addupdate_scatter(t, [dlb + cols[f]], val)
                        return _
                    lax.fori_loop(0, 8, grp, None)
                return _
            lax.fori_loop(0, NCH_Q, chunk, None)

        off_hi = pl.multiple_of(u * (THI_ROWS * GF), 8)
        pltpu.sync_copy(t_hi.at[pl.ds(0, THI_ROWS * GF)],
                        o_hi.at[pl.ds(off_hi, THI_ROWS * GF)])
        off_af = pl.multiple_of(u * (TAF_ROWS * GF), 8)
        pltpu.sync_copy(t_af.at[pl.ds(0, TAF_ROWS * GF)],
                        o_af.at[pl.ds(off_af, TAF_ROWS * GF)])

    o_hi, o_af = k(d_hi_r, d_af_r, hi_src, hi_dst, af_src, af_dst)
    o_hi = o_hi.reshape(4, NG, THI_ROWS, GF).transpose(0, 2, 1, 3)
    o_af = o_af.reshape(4, NG, TAF_ROWS, GF).transpose(0, 2, 1, 3)
    return (o_hi.reshape(4, THI_ROWS, H)[:, :NI],
            o_af.reshape(4, TAF_ROWS, H)[:, :NF])


def _dsi(deg):
    return jnp.where(deg > 0, jax.lax.rsqrt(jnp.maximum(deg, 1e-12)), 0.0)


def _enc_body(x_ref, w_ref, b_ref, hhi_ref, haf_ref, out_hi_ref, out_af_ref):
    d = jax.nn.relu(jnp.dot(x_ref[...], w_ref[...],
                            preferred_element_type=jnp.float32,
                            precision=jax.lax.Precision.HIGHEST) + b_ref[...])
    dsi_hi = _dsi(jnp.sum(hhi_ref[...], axis=0))
    dsi_af = _dsi(jnp.sum(haf_ref[...], axis=0))
    out_hi_ref[...] = d * dsi_hi[:, None]
    out_af_ref[...] = d * dsi_af[:, None]


def _encode_dealers(x_dealer, W, b, hs_hi, hs_af):
    TR = 2048
    grid = (pl.cdiv(ND, TR),)
    return pl.pallas_call(
        _enc_body,
        grid=grid,
        in_specs=[
            pl.BlockSpec((TR, 64), lambda i: (i, 0)),
            pl.BlockSpec((64, H), lambda i: (0, 0)),
            pl.BlockSpec((1, H), lambda i: (0, 0)),
            pl.BlockSpec((32, TR), lambda i: (0, i)),
            pl.BlockSpec((32, TR), lambda i: (0, i)),
        ],
        out_specs=[pl.BlockSpec((TR, H), lambda i: (i, 0))] * 2,
        out_shape=[jax.ShapeDtypeStruct((ND, H), jnp.float32)] * 2,
        interpret=_INTERPRET,
    )(x_dealer, W, b.reshape(1, H), hs_hi, hs_af)


def _small_body(xi_ref, wei_ref, bei_ref,
                hdhi_ref, hsrb_ref, hdrb_ref, hdaf_ref, cnt_ref,
                agghi_ref, aggaf_ref,
                w1hi_ref, b1hi_ref, w2hi_ref, b2hi_ref,
                w2rb_ref, b2rb_ref, w2af_ref, b2af_ref,
                wp1a_ref, wp1b_ref, bp1_ref,
                a_ref, bmat_ref):
    f32 = jnp.float32
    hp = jax.lax.Precision.HIGHEST
    ddi_hi = _dsi(jnp.sum(hdhi_ref[...], axis=0)[:NI])
    dsi_rb = _dsi(jnp.sum(hsrb_ref[...], axis=0)[:NI])
    ddi_rb = _dsi(jnp.sum(hdrb_ref[...], axis=0)[:NF])
    ddi_af = _dsi(jnp.sum(hdaf_ref[...], axis=0)[:NF])

    agg_hi = jnp.sum(agghi_ref[...], axis=0)
    agg_af = jnp.sum(aggaf_ref[...], axis=0)

    ii = jax.nn.relu(jnp.dot(xi_ref[...], wei_ref[...],
                             preferred_element_type=f32, precision=hp)
                     + bei_ref[...])
    M = jnp.sum(cnt_ref[...], axis=0) * dsi_rb[None, :]

    i1 = ddi_hi[:, None] * jnp.dot(agg_hi, w1hi_ref[...],
                                   preferred_element_type=f32, precision=hp) \
        + b1hi_ref[...]
    i2 = ddi_hi[:, None] * jnp.dot(agg_hi, w2hi_ref[...],
                                   preferred_element_type=f32, precision=hp) \
        + b2hi_ref[...]
    i1r = jax.nn.relu(i1)
    g2 = jnp.dot(M, i1r, preferred_element_type=f32, precision=hp)
    f2 = (ddi_rb[:, None] * jnp.dot(g2, w2rb_ref[...],
                                    preferred_element_type=f32, precision=hp)
          + b2rb_ref[...]
          + ddi_af[:, None] * jnp.dot(agg_af, w2af_ref[...],
                                      preferred_element_type=f32, precision=hp)
          + b2af_ref[...])

    a_ref[...] = jnp.dot(i2, wp1a_ref[...], preferred_element_type=f32,
                         precision=hp)
    bmat_ref[...] = jnp.dot(f2, wp1b_ref[...], preferred_element_type=f32,
                            precision=hp) + bp1_ref[...]


def _small_stage(x_intent, W_enc_i, b_enc_i, hd_hi, hs_rb, hd_rb, hd_af, cnt,
                 agg_hi_p, agg_af_p, W1_hi, b1_hi, W2_hi, b2_hi,
                 W2_rb, b2_rb, W2_af, b2_af, Wp1, bp1):
    return pl.pallas_call(
        _small_body,
        out_shape=[
            jax.ShapeDtypeStruct((NI, H), jnp.float32),
            jax.ShapeDtypeStruct((NF, H), jnp.float32),
        ],
        interpret=_INTERPRET,
    )(x_intent, W_enc_i, b_enc_i.reshape(1, H),
      hd_hi, hs_rb, hd_rb, hd_af, cnt,
      agg_hi_p, agg_af_p,
      W1_hi, b1_hi.reshape(1, H), W2_hi, b2_hi.reshape(1, H),
      W2_rb, b2_rb.reshape(1, H), W2_af, b2_af.reshape(1, H),
      Wp1[:H], Wp1[H:], bp1.reshape(1, H))


def _pair_body(a_ref, b_ref, wp2_ref, bp2_ref, wp3_ref, bp3_ref, out_ref, *, tr):
    f32 = jnp.float32
    hp = jax.lax.Precision.HIGHEST
    h1 = jax.nn.relu(a_ref[...][:, None, :] + b_ref[...][None, :, :])
    h1 = h1.reshape(tr * NF, H)
    h2 = jax.nn.relu(jnp.dot(h1, wp2_ref[...], preferred_element_type=f32,
                             precision=hp) + bp2_ref[...])
    logit = jnp.dot(h2, wp3_ref[...], preferred_element_type=f32,
                    precision=hp) + bp3_ref[...]
    out_ref[...] = jax.nn.sigmoid(logit.reshape(tr, NF))


def _pairwise(A, B, Wp2, bp2, Wp3, bp3):
    TR = 200
    grid = (NI // TR,)
    return pl.pallas_call(
        functools.partial(_pair_body, tr=TR),
        grid=grid,
        in_specs=[
            pl.BlockSpec((TR, H), lambda i: (i, 0)),
            pl.BlockSpec((NF, H), lambda i: (0, 0)),
            pl.BlockSpec((H, H // 2), lambda i: (0, 0)),
            pl.BlockSpec((1, H // 2), lambda i: (0, 0)),
            pl.BlockSpec((H // 2, 1), lambda i: (0, 0)),
            pl.BlockSpec((1, 1), lambda i: (0, 0)),
        ],
        out_specs=pl.BlockSpec((TR, NF), lambda i: (i, 0)),
        out_shape=jax.ShapeDtypeStruct((NI, NF), jnp.float32),
        interpret=_INTERPRET,
    )(A, B, Wp2, bp2.reshape(1, H // 2), Wp3, bp3.reshape(1, 1))


def kernel(x_dealer, x_intent, x_fix, edge_has_intent, edge_resolved_by,
           edge_applies_fix, W_enc_d, b_enc_d, W_enc_i, b_enc_i, W_enc_f,
           b_enc_f, W1_hi, b1_hi, W1_rb, b1_rb, W1_af, b1_af, W2_hi, b2_hi,
           W2_rb, b2_rb, W2_af, b2_af, Wp1, bp1, Wp2, bp2, Wp3, bp3):
    ehi, erb, eaf = edge_has_intent, edge_resolved_by, edge_applies_fix

    flat_rb = erb[1] * NI + erb[0]
    hs_hi, hd_hi, hs_af, hd_af, hs_rb, hd_rb, cnt = _sc_hist(
        _pad1(ehi[0], PAD_BIG, ND), _pad1(ehi[1], PAD_BIG, NI),
        _pad1(eaf[0], PAD_BIG, ND), _pad1(eaf[1], PAD_BIG, NF),
        _pad1(erb[0], PAD_RB, NI), _pad1(erb[1], PAD_RB, NF),
        _pad1(flat_rb, PAD_RB, NF * NI + NI))
    cnt = cnt[:, :NF * NI].reshape(32, NF, NI)

    d_hi, d_af = _encode_dealers(x_dealer, W_enc_d, b_enc_d, hs_hi, hs_af)

    agg_hi_p, agg_af_p = _sc_agg(
        d_hi.reshape(ND * NG, GF), d_af.reshape(ND * NG, GF),
        _pad1(ehi[0], PAD_Q, 0), _pad1(ehi[1], PAD_Q, THI_ROWS - 1),
        _pad1(eaf[0], PAD_Q, 0), _pad1(eaf[1], PAD_Q, TAF_ROWS - 1))

    A, B = _small_stage(x_intent, W_enc_i, b_enc_i, hd_hi, hs_rb, hd_rb, hd_af,
                        cnt, agg_hi_p, agg_af_p, W1_hi, b1_hi, W2_hi, b2_hi,
                        W2_rb, b2_rb, W2_af, b2_af, Wp1, bp1)
    return _pairwise(A, B, Wp2, bp2, Wp3, bp3)


# speedup vs baseline: 3.3378x; 1.0431x over previous
import functools

import jax
import jax.numpy as jnp
from jax import lax
from jax.experimental import pallas as pl
from jax.experimental.pallas import tpu as pltpu
from jax.experimental.pallas import tpu_sc as plsc

ND, NI, NF = 100000, 2000, 64
H = 256
E_HI, E_RB, E_AF = 200000, 64000, 200000

NPD = 100352
T_SM, T_F = 2048, 256
T_CNT_H = 65024
CH = 2048
PAD_BIG = 200704
PAD_RB = 65536
PAD_Q = 204800
NCH_Q = 25
GF = 32
NG = 8
THI_ROWS = 2048
TAF_ROWS = 128

_INTERPRET = False


def _pad1(a, tot, fill):
    return jnp.pad(a, (0, tot - a.shape[0]), constant_values=jnp.int32(fill))


def _sc_hist(h_ehi0, h_ehi1, h_eaf0, h_eaf1, h_erb0, h_erb1, h_flat):
    f32 = jnp.float32
    i32 = jnp.int32
    mesh = plsc.VectorSubcoreMesh(core_axis_name="c", subcore_axis_name="s")

    @functools.partial(
        pl.kernel,
        out_type=[
            jax.ShapeDtypeStruct((32 * NPD,), f32),
            jax.ShapeDtypeStruct((32 * T_SM,), f32),
            jax.ShapeDtypeStruct((32 * NPD,), f32),
            jax.ShapeDtypeStruct((32 * T_F,), f32),
            jax.ShapeDtypeStruct((32 * T_SM,), f32),
            jax.ShapeDtypeStruct((32 * T_F,), f32),
            jax.ShapeDtypeStruct((32 * 2 * T_CNT_H,), f32),
        ],
        mesh=mesh,
        scratch_types=[
            pltpu.VMEM((NPD,), f32),
            pltpu.VMEM((PAD_BIG // 32,), i32),
        ],
        compiler_params=pltpu.CompilerParams(needs_layout_passes=False),
    )
    def k(e0_h, e1_h, a0_h, a1_h, r0_h, r1_h, fl_h,
          o0, o1, o2, o3, o4, o5, o6,
          tb, eb):
        c = lax.axis_index("c")
        s = lax.axis_index("s")
        u = c * 16 + s
        iota = lax.iota(i32, 16)
        lmask = [iota == l for l in range(16)]
        ones16 = jnp.ones((16,), f32)

        phases = (
            (e0_h, PAD_BIG, NPD, o0, NPD, 0, False),
            (e1_h, PAD_BIG, T_SM, o1, T_SM, 0, False),
            (a0_h, PAD_BIG, NPD, o2, NPD, 0, False),
            (a1_h, PAD_BIG, T_F, o3, T_F, 0, False),
            (r0_h, PAD_RB, T_SM, o4, T_SM, 0, False),
            (r1_h, PAD_RB, T_F, o5, T_F, 0, False),
            (fl_h, PAD_RB, T_CNT_H, o6, 2 * T_CNT_H, 0, True),
            (fl_h, PAD_RB, T_CNT_H, o6, 2 * T_CNT_H, T_CNT_H, True),
        )
        for arr_h, epad, nbins, out, ostride, binlo, ranged in phases:
            epp = epad // 32

            def zt(j, _):
                tb[pl.ds(j * 16, 16)] = jnp.zeros((16,), f32)
                return _
            lax.fori_loop(0, nbins // 16, zt, None)

            pltpu.sync_copy(arr_h.at[pl.ds(u * epp, epp)], eb.at[pl.ds(0, epp)])

            def sv(j, _, nbins=nbins, binlo=binlo, ranged=ranged):
                v = eb[pl.ds(j * 16, 16)] - binlo
                if ranged:
                    m = (v >= 0) & (v < nbins)
                    for l in range(16):
                        plsc.addupdate_scatter(tb, [v], ones16,
                                               mask=m & lmask[l])
                else:
                    for l in range(16):
                        plsc.addupdate_scatter(tb, [v], ones16, mask=lmask[l])
                return _
            lax.fori_loop(0, epp // 16, sv, None)

            off = pl.multiple_of(u * ostride + binlo, 8)
            pltpu.sync_copy(tb.at[pl.ds(0, nbins)], out.at[pl.ds(off, nbins)])

    outs = k(h_ehi0, h_ehi1, h_eaf0, h_eaf1, h_erb0, h_erb1, h_flat)
    return [o.reshape(32, -1) for o in outs]


def _sc_agg(d_hi_r, d_af_r, hi_src, hi_dst, af_src, af_dst):
    f32 = jnp.float32
    i32 = jnp.int32
    mesh = plsc.VectorSubcoreMesh(core_axis_name="c", subcore_axis_name="s")

    @functools.partial(
        pl.kernel,
        out_type=[
            jax.ShapeDtypeStruct((32 * THI_ROWS * GF,), f32),
            jax.ShapeDtypeStruct((32 * TAF_ROWS * GF,), f32),
        ],
        mesh=mesh,
        scratch_types=[
            pltpu.VMEM((THI_ROWS * GF,), f32),
            pltpu.VMEM((TAF_ROWS * GF,), f32),
            pltpu.VMEM((CH,), i32),
            pltpu.VMEM((CH,), i32),
            pltpu.VMEM((128, GF), f32),
            pltpu.VMEM((128, GF), f32),
            pltpu.VMEM((128,), i32),
            pltpu.VMEM((128,), i32),
            pltpu.SemaphoreType.DMA,
            pltpu.SemaphoreType.DMA,
        ],
        compiler_params=pltpu.CompilerParams(needs_layout_passes=False,
                                             use_tc_tiling_on_sc=False),
    )
    def k(dh_h, da_h, hs_h, hd_h, as_h, ad_h,
          o_hi, o_af,
          t_hi, t_af, sbuf, dbuf, rows0, rows1, idxs0, idxs1, sem0, sem1):
        c = lax.axis_index("c")
        s = lax.axis_index("s")
        u = c * 16 + s
        q = u // NG
        g = u % NG
        iota = lax.iota(i32, 16)
        cols = [f * 16 + iota for f in range(GF // 16)]
        rows_b = (rows0, rows1)
        idxs_b = (idxs0, idxs1)
        sems = (sem0, sem1)

        for t, n in ((t_hi, THI_ROWS * GF), (t_af, TAF_ROWS * GF)):
            def zt(j, _, t=t):
                t[pl.ds(j * 16, 16)] = jnp.zeros((16,), f32)
                return _
            lax.fori_loop(0, n // 16, zt, None)

        for src_h, dst_h, d_h, t in ((hs_h, hd_h, dh_h, t_hi),
                                     (as_h, ad_h, da_h, t_af)):
            def chunk(kk, _, src_h=src_h, dst_h=dst_h, d_h=d_h, t=t):
                base = pl.multiple_of(q * (NCH_Q * CH) + kk * CH, 8)
                pltpu.sync_copy(src_h.at[pl.ds(base, CH)], sbuf)
                pltpu.sync_copy(dst_h.at[pl.ds(base, CH)], dbuf)

                def xf(j, _):
                    sbuf[pl.ds(j * 16, 16)] = sbuf[pl.ds(j * 16, 16)] * NG + g
                    dbuf[pl.ds(j * 16, 16)] = dbuf[pl.ds(j * 16, 16)] * GF
                    return _
                lax.fori_loop(0, CH // 16, xf, None)

                def bloop(bb, _, d_h=d_h, t=t):
                    for qq in range(8):
                        idxs0[pl.ds(qq * 16, 16)] = \
                            sbuf[pl.ds(bb * 128 + qq * 16, 16)]
                    pltpu.async_copy(d_h.at[idxs0], rows0, sem0).wait()

                    @plsc.parallel_loop(0, 8)
                    def grp(gr, t=t, bb=bb):
                        for l in range(16):
                            posv = jnp.broadcast_to(bb * 128 + gr * 16 + l,
                                                    (16,)).astype(i32)
                            dlb = plsc.load_gather(dbuf, [posv])
                            erow = posv - bb * 128
                            for f in range(GF // 16):
                                val = plsc.load_gather(rows0, [erow, cols[f]])
                                plsc.addupdate_scatter(t, [dlb + cols[f]], val)
                    return _
                lax.fori_loop(0, CH // 128, bloop, None)
                return _
            lax.fori_loop(0, NCH_Q, chunk, None)

        off_hi = pl.multiple_of(u * (THI_ROWS * GF), 8)
        pltpu.sync_copy(t_hi.at[pl.ds(0, THI_ROWS * GF)],
                        o_hi.at[pl.ds(off_hi, THI_ROWS * GF)])
        off_af = pl.multiple_of(u * (TAF_ROWS * GF), 8)
        pltpu.sync_copy(t_af.at[pl.ds(0, TAF_ROWS * GF)],
                        o_af.at[pl.ds(off_af, TAF_ROWS * GF)])

    o_hi, o_af = k(d_hi_r, d_af_r, hi_src, hi_dst, af_src, af_dst)
    o_hi = o_hi.reshape(4, NG, THI_ROWS, GF).transpose(0, 2, 1, 3)
    o_af = o_af.reshape(4, NG, TAF_ROWS, GF).transpose(0, 2, 1, 3)
    return (o_hi.reshape(4, THI_ROWS, H)[:, :NI],
            o_af.reshape(4, TAF_ROWS, H)[:, :NF])


def _dsi(deg):
    return jnp.where(deg > 0, jax.lax.rsqrt(jnp.maximum(deg, 1e-12)), 0.0)


def _enc_body(x_ref, w_ref, b_ref, hhi_ref, haf_ref, out_hi_ref, out_af_ref):
    d = jax.nn.relu(jnp.dot(x_ref[...], w_ref[...],
                            preferred_element_type=jnp.float32,
                            precision=jax.lax.Precision.HIGHEST) + b_ref[...])
    dsi_hi = _dsi(jnp.sum(hhi_ref[...], axis=0))
    dsi_af = _dsi(jnp.sum(haf_ref[...], axis=0))
    out_hi_ref[...] = d * dsi_hi[:, None]
    out_af_ref[...] = d * dsi_af[:, None]


def _encode_dealers(x_dealer, W, b, hs_hi, hs_af):
    TR = 2048
    grid = (pl.cdiv(ND, TR),)
    return pl.pallas_call(
        _enc_body,
        grid=grid,
        in_specs=[
            pl.BlockSpec((TR, 64), lambda i: (i, 0)),
            pl.BlockSpec((64, H), lambda i: (0, 0)),
            pl.BlockSpec((1, H), lambda i: (0, 0)),
            pl.BlockSpec((32, TR), lambda i: (0, i)),
            pl.BlockSpec((32, TR), lambda i: (0, i)),
        ],
        out_specs=[pl.BlockSpec((TR, H), lambda i: (i, 0))] * 2,
        out_shape=[jax.ShapeDtypeStruct((ND, H), jnp.float32)] * 2,
        interpret=_INTERPRET,
    )(x_dealer, W, b.reshape(1, H), hs_hi, hs_af)


def _small_body(xi_ref, wei_ref, bei_ref,
                hdhi_ref, hsrb_ref, hdrb_ref, hdaf_ref, cnt_ref,
                agghi_ref, aggaf_ref,
                w1hi_ref, b1hi_ref, w2hi_ref, b2hi_ref,
                w2rb_ref, b2rb_ref, w2af_ref, b2af_ref,
                wp1a_ref, wp1b_ref, bp1_ref,
                a_ref, bmat_ref):
    f32 = jnp.float32
    hp = jax.lax.Precision.HIGHEST
    ddi_hi = _dsi(jnp.sum(hdhi_ref[...], axis=0)[:NI])
    dsi_rb = _dsi(jnp.sum(hsrb_ref[...], axis=0)[:NI])
    ddi_rb = _dsi(jnp.sum(hdrb_ref[...], axis=0)[:NF])
    ddi_af = _dsi(jnp.sum(hdaf_ref[...], axis=0)[:NF])

    agg_hi = jnp.sum(agghi_ref[...], axis=0)
    agg_af = jnp.sum(aggaf_ref[...], axis=0)

    ii = jax.nn.relu(jnp.dot(xi_ref[...], wei_ref[...],
                             preferred_element_type=f32, precision=hp)
                     + bei_ref[...])
    M = jnp.sum(cnt_ref[...], axis=0) * dsi_rb[None, :]

    i1 = ddi_hi[:, None] * jnp.dot(agg_hi, w1hi_ref[...],
                                   preferred_element_type=f32, precision=hp) \
        + b1hi_ref[...]
    i2 = ddi_hi[:, None] * jnp.dot(agg_hi, w2hi_ref[...],
                                   preferred_element_type=f32, precision=hp) \
        + b2hi_ref[...]
    i1r = jax.nn.relu(i1)
    g2 = jnp.dot(M, i1r, preferred_element_type=f32, precision=hp)
    f2 = (ddi_rb[:, None] * jnp.dot(g2, w2rb_ref[...],
                                    preferred_element_type=f32, precision=hp)
          + b2rb_ref[...]
          + ddi_af[:, None] * jnp.dot(agg_af, w2af_ref[...],
                                      preferred_element_type=f32, precision=hp)
          + b2af_ref[...])

    a_ref[...] = jnp.dot(i2, wp1a_ref[...], preferred_element_type=f32,
                         precision=hp)
    bmat_ref[...] = jnp.dot(f2, wp1b_ref[...], preferred_element_type=f32,
                            precision=hp) + bp1_ref[...]


def _small_stage(x_intent, W_enc_i, b_enc_i, hd_hi, hs_rb, hd_rb, hd_af, cnt,
                 agg_hi_p, agg_af_p, W1_hi, b1_hi, W2_hi, b2_hi,
                 W2_rb, b2_rb, W2_af, b2_af, Wp1, bp1):
    return pl.pallas_call(
        _small_body,
        out_shape=[
            jax.ShapeDtypeStruct((NI, H), jnp.float32),
            jax.ShapeDtypeStruct((NF, H), jnp.float32),
        ],
        interpret=_INTERPRET,
    )(x_intent, W_enc_i, b_enc_i.reshape(1, H),
      hd_hi, hs_rb, hd_rb, hd_af, cnt,
      agg_hi_p, agg_af_p,
      W1_hi, b1_hi.reshape(1, H), W2_hi, b2_hi.reshape(1, H),
      W2_rb, b2_rb.reshape(1, H), W2_af, b2_af.reshape(1, H),
      Wp1[:H], Wp1[H:], bp1.reshape(1, H))


def _pair_body(a_ref, b_ref, wp2_ref, bp2_ref, wp3_ref, bp3_ref, out_ref, *, tr):
    f32 = jnp.float32
    hp = jax.lax.Precision.HIGHEST
    h1 = jax.nn.relu(a_ref[...][:, None, :] + b_ref[...][None, :, :])
    h1 = h1.reshape(tr * NF, H)
    h2 = jax.nn.relu(jnp.dot(h1, wp2_ref[...], preferred_element_type=f32,
                             precision=hp) + bp2_ref[...])
    logit = jnp.dot(h2, wp3_ref[...], preferred_element_type=f32,
                    precision=hp) + bp3_ref[...]
    out_ref[...] = jax.nn.sigmoid(logit.reshape(tr, NF))


def _pairwise(A, B, Wp2, bp2, Wp3, bp3):
    TR = 200
    grid = (NI // TR,)
    return pl.pallas_call(
        functools.partial(_pair_body, tr=TR),
        grid=grid,
        in_specs=[
            pl.BlockSpec((TR, H), lambda i: (i, 0)),
            pl.BlockSpec((NF, H), lambda i: (0, 0)),
            pl.BlockSpec((H, H // 2), lambda i: (0, 0)),
            pl.BlockSpec((1, H // 2), lambda i: (0, 0)),
            pl.BlockSpec((H // 2, 1), lambda i: (0, 0)),
            pl.BlockSpec((1, 1), lambda i: (0, 0)),
        ],
        out_specs=pl.BlockSpec((TR, NF), lambda i: (i, 0)),
        out_shape=jax.ShapeDtypeStruct((NI, NF), jnp.float32),
        interpret=_INTERPRET,
    )(A, B, Wp2, bp2.reshape(1, H // 2), Wp3, bp3.reshape(1, 1))


def kernel(x_dealer, x_intent, x_fix, edge_has_intent, edge_resolved_by,
           edge_applies_fix, W_enc_d, b_enc_d, W_enc_i, b_enc_i, W_enc_f,
           b_enc_f, W1_hi, b1_hi, W1_rb, b1_rb, W1_af, b1_af, W2_hi, b2_hi,
           W2_rb, b2_rb, W2_af, b2_af, Wp1, bp1, Wp2, bp2, Wp3, bp3):
    ehi, erb, eaf = edge_has_intent, edge_resolved_by, edge_applies_fix

    flat_rb = erb[1] * NI + erb[0]
    hs_hi, hd_hi, hs_af, hd_af, hs_rb, hd_rb, cnt = _sc_hist(
        _pad1(ehi[0], PAD_BIG, ND), _pad1(ehi[1], PAD_BIG, NI),
        _pad1(eaf[0], PAD_BIG, ND), _pad1(eaf[1], PAD_BIG, NF),
        _pad1(erb[0], PAD_RB, NI), _pad1(erb[1], PAD_RB, NF),
        _pad1(flat_rb, PAD_RB, NF * NI + NI))
    cnt = cnt[:, :NF * NI].reshape(32, NF, NI)

    d_hi, d_af = _encode_dealers(x_dealer, W_enc_d, b_enc_d, hs_hi, hs_af)

    agg_hi_p, agg_af_p = _sc_agg(
        d_hi.reshape(ND * NG, GF), d_af.reshape(ND * NG, GF),
        _pad1(ehi[0], PAD_Q, 0), _pad1(ehi[1], PAD_Q, THI_ROWS - 1),
        _pad1(eaf[0], PAD_Q, 0), _pad1(eaf[1], PAD_Q, TAF_ROWS - 1))

    A, B = _small_stage(x_intent, W_enc_i, b_enc_i, hd_hi, hs_rb, hd_rb, hd_af,
                        cnt, agg_hi_p, agg_af_p, W1_hi, b1_hi, W2_hi, b2_hi,
                        W2_rb, b2_rb, W2_af, b2_af, Wp1, bp1)
    return _pairwise(A, B, Wp2, bp2, Wp3, bp3)


# speedup vs baseline: 3.7568x; 1.1255x over previous
import functools

import jax
import jax.numpy as jnp
from jax import lax
from jax.experimental import pallas as pl
from jax.experimental.pallas import tpu as pltpu
from jax.experimental.pallas import tpu_sc as plsc

ND, NI, NF = 100000, 2000, 64
H = 256
E_HI, E_RB, E_AF = 200000, 64000, 200000

NPD = 100352
T_SM, T_F = 2048, 256
T_CNT_H = 65024
CH = 2048
PAD_BIG = 200704
PAD_RB = 65536
PAD_Q = 204800
NCH_Q = 25
GF = 32
NG = 8
THI_ROWS = 2048
TAF_ROWS = 128

_INTERPRET = False


def _pad1(a, tot, fill):
    return jnp.pad(a, (0, tot - a.shape[0]), constant_values=jnp.int32(fill))


def _sc_hist(h_ehi0, h_ehi1, h_eaf0, h_eaf1, h_erb0, h_erb1, h_flat):
    f32 = jnp.float32
    i32 = jnp.int32
    mesh = plsc.VectorSubcoreMesh(core_axis_name="c", subcore_axis_name="s")

    @functools.partial(
        pl.kernel,
        out_type=[
            jax.ShapeDtypeStruct((32 * NPD,), f32),
            jax.ShapeDtypeStruct((32 * T_SM,), f32),
            jax.ShapeDtypeStruct((32 * NPD,), f32),
            jax.ShapeDtypeStruct((32 * T_F,), f32),
            jax.ShapeDtypeStruct((32 * T_SM,), f32),
            jax.ShapeDtypeStruct((32 * T_F,), f32),
            jax.ShapeDtypeStruct((32 * 2 * T_CNT_H,), f32),
        ],
        mesh=mesh,
        scratch_types=[
            pltpu.VMEM((NPD,), f32),
            pltpu.VMEM((PAD_BIG // 32,), i32),
        ],
        compiler_params=pltpu.CompilerParams(needs_layout_passes=False),
    )
    def k(e0_h, e1_h, a0_h, a1_h, r0_h, r1_h, fl_h,
          o0, o1, o2, o3, o4, o5, o6,
          tb, eb):
        c = lax.axis_index("c")
        s = lax.axis_index("s")
        u = c * 16 + s
        iota = lax.iota(i32, 16)
        lmask = [iota == l for l in range(16)]
        ones16 = jnp.ones((16,), f32)

        phases = (
            (e0_h, PAD_BIG, NPD, o0, NPD, 0, False),
            (e1_h, PAD_BIG, T_SM, o1, T_SM, 0, False),
            (a0_h, PAD_BIG, NPD, o2, NPD, 0, False),
            (a1_h, PAD_BIG, T_F, o3, T_F, 0, False),
            (r0_h, PAD_RB, T_SM, o4, T_SM, 0, False),
            (r1_h, PAD_RB, T_F, o5, T_F, 0, False),
            (fl_h, PAD_RB, T_CNT_H, o6, 2 * T_CNT_H, 0, True),
            (fl_h, PAD_RB, T_CNT_H, o6, 2 * T_CNT_H, T_CNT_H, True),
        )
        for arr_h, epad, nbins, out, ostride, binlo, ranged in phases:
            epp = epad // 32

            def zt(j, _):
                tb[pl.ds(j * 16, 16)] = jnp.zeros((16,), f32)
                return _
            lax.fori_loop(0, nbins // 16, zt, None)

            pltpu.sync_copy(arr_h.at[pl.ds(u * epp, epp)], eb.at[pl.ds(0, epp)])

            def sv(j, _, nbins=nbins, binlo=binlo, ranged=ranged):
                v = eb[pl.ds(j * 16, 16)] - binlo
                if ranged:
                    m = (v >= 0) & (v < nbins)
                    for l in range(16):
                        plsc.addupdate_scatter(tb, [v], ones16,
                                               mask=m & lmask[l])
                else:
                    for l in range(16):
                        plsc.addupdate_scatter(tb, [v], ones16, mask=lmask[l])
                return _
            lax.fori_loop(0, epp // 16, sv, None)

            off = pl.multiple_of(u * ostride + binlo, 8)
            pltpu.sync_copy(tb.at[pl.ds(0, nbins)], out.at[pl.ds(off, nbins)])

    outs = k(h_ehi0, h_ehi1, h_eaf0, h_eaf1, h_erb0, h_erb1, h_flat)
    return [o.reshape(32, -1) for o in outs]


def _sc_agg(d_hi_r, d_af_r, hi_src, hi_dst, af_src, af_dst):
    f32 = jnp.float32
    i32 = jnp.int32
    mesh = plsc.VectorSubcoreMesh(core_axis_name="c", subcore_axis_name="s")

    @functools.partial(
        pl.kernel,
        out_type=[
            jax.ShapeDtypeStruct((32 * THI_ROWS * GF,), f32),
            jax.ShapeDtypeStruct((32 * TAF_ROWS * GF,), f32),
        ],
        mesh=mesh,
        scratch_types=[
            pltpu.VMEM((THI_ROWS * GF,), f32),
            pltpu.VMEM((TAF_ROWS * GF,), f32),
            pltpu.VMEM((CH,), i32),
            pltpu.VMEM((CH,), i32),
            pltpu.VMEM((128, GF), f32),
            pltpu.VMEM((128, GF), f32),
            pltpu.VMEM((128,), i32),
            pltpu.VMEM((128,), i32),
            pltpu.SemaphoreType.DMA,
            pltpu.SemaphoreType.DMA,
        ],
        compiler_params=pltpu.CompilerParams(needs_layout_passes=False,
                                             use_tc_tiling_on_sc=False),
    )
    def k(dh_h, da_h, hs_h, hd_h, as_h, ad_h,
          o_hi, o_af,
          t_hi, t_af, sbuf, dbuf, rows0, rows1, idxs0, idxs1, sem0, sem1):
        c = lax.axis_index("c")
        s = lax.axis_index("s")
        u = c * 16 + s
        q = u // NG
        g = u % NG
        iota = lax.iota(i32, 16)
        cols = [f * 16 + iota for f in range(GF // 16)]
        rows_b = (rows0, rows1)
        idxs_b = (idxs0, idxs1)
        sems = (sem0, sem1)

        for t, n in ((t_hi, THI_ROWS * GF), (t_af, TAF_ROWS * GF)):
            def zt(j, _, t=t):
                t[pl.ds(j * 16, 16)] = jnp.zeros((16,), f32)
                return _
            lax.fori_loop(0, n // 16, zt, None)

        for src_h, dst_h, d_h, t in ((hs_h, hd_h, dh_h, t_hi),
                                     (as_h, ad_h, da_h, t_af)):
            def chunk(kk, _, src_h=src_h, dst_h=dst_h, d_h=d_h, t=t):
                base = pl.multiple_of(q * (NCH_Q * CH) + kk * CH, 8)
                pltpu.sync_copy(src_h.at[pl.ds(base, CH)], sbuf)
                pltpu.sync_copy(dst_h.at[pl.ds(base, CH)], dbuf)

                def xf(j, _):
                    sbuf[pl.ds(j * 16, 16)] = sbuf[pl.ds(j * 16, 16)] * NG + g
                    dbuf[pl.ds(j * 16, 16)] = dbuf[pl.ds(j * 16, 16)] * GF
                    return _
                lax.fori_loop(0, CH // 16, xf, None)

                def acc(bb, rr, t=t):
                    @plsc.parallel_loop(0, 8)
                    def grp(gr, t=t, bb=bb, rr=rr):
                        for l in range(16):
                            posv = jnp.broadcast_to(bb * 128 + gr * 16 + l,
                                                    (16,)).astype(i32)
                            dlb = plsc.load_gather(dbuf, [posv])
                            erow = posv - bb * 128
                            for f in range(GF // 16):
                                val = plsc.load_gather(rr, [erow, cols[f]])
                                plsc.addupdate_scatter(t, [dlb + cols[f]], val)

                def stage(bb, ib, rb, sm, d_h=d_h):
                    for qq in range(8):
                        ib[pl.ds(qq * 16, 16)] = \
                            sbuf[pl.ds(bb * 128 + qq * 16, 16)]
                    pltpu.async_copy(d_h.at[ib], rb, sm)

                def bloop(bp, _, d_h=d_h, t=t):
                    stage(2 * bp, idxs0, rows0, sem0)
                    stage(2 * bp + 1, idxs1, rows1, sem1)
                    pltpu.make_async_copy(d_h.at[idxs0], rows0, sem0).wait()
                    pltpu.make_async_copy(d_h.at[idxs1], rows1, sem1).wait()
                    acc(2 * bp, rows0)
                    acc(2 * bp + 1, rows1)
                    return _
                lax.fori_loop(0, CH // 256, bloop, None)
                return _
            lax.fori_loop(0, NCH_Q, chunk, None)

        off_hi = pl.multiple_of(u * (THI_ROWS * GF), 8)
        pltpu.sync_copy(t_hi.at[pl.ds(0, THI_ROWS * GF)],
                        o_hi.at[pl.ds(off_hi, THI_ROWS * GF)])
        off_af = pl.multiple_of(u * (TAF_ROWS * GF), 8)
        pltpu.sync_copy(t_af.at[pl.ds(0, TAF_ROWS * GF)],
                        o_af.at[pl.ds(off_af, TAF_ROWS * GF)])

    o_hi, o_af = k(d_hi_r, d_af_r, hi_src, hi_dst, af_src, af_dst)
    o_hi = o_hi.reshape(4, NG, THI_ROWS, GF).transpose(0, 2, 1, 3)
    o_af = o_af.reshape(4, NG, TAF_ROWS, GF).transpose(0, 2, 1, 3)
    return (o_hi.reshape(4, THI_ROWS, H)[:, :NI],
            o_af.reshape(4, TAF_ROWS, H)[:, :NF])


def _dsi(deg):
    return jnp.where(deg > 0, jax.lax.rsqrt(jnp.maximum(deg, 1e-12)), 0.0)


def _enc_body(x_ref, w_ref, b_ref, hhi_ref, haf_ref, out_hi_ref, out_af_ref):
    d = jax.nn.relu(jnp.dot(x_ref[...], w_ref[...],
                            preferred_element_type=jnp.float32,
                            precision=jax.lax.Precision.HIGHEST) + b_ref[...])
    dsi_hi = _dsi(jnp.sum(hhi_ref[...], axis=0))
    dsi_af = _dsi(jnp.sum(haf_ref[...], axis=0))
    out_hi_ref[...] = d * dsi_hi[:, None]
    out_af_ref[...] = d * dsi_af[:, None]


def _encode_dealers(x_dealer, W, b, hs_hi, hs_af):
    TR = 2048
    grid = (pl.cdiv(ND, TR),)
    return pl.pallas_call(
        _enc_body,
        grid=grid,
        in_specs=[
            pl.BlockSpec((TR, 64), lambda i: (i, 0)),
            pl.BlockSpec((64, H), lambda i: (0, 0)),
            pl.BlockSpec((1, H), lambda i: (0, 0)),
            pl.BlockSpec((32, TR), lambda i: (0, i)),
            pl.BlockSpec((32, TR), lambda i: (0, i)),
        ],
        out_specs=[pl.BlockSpec((TR, H), lambda i: (i, 0))] * 2,
        out_shape=[jax.ShapeDtypeStruct((ND, H), jnp.float32)] * 2,
        interpret=_INTERPRET,
    )(x_dealer, W, b.reshape(1, H), hs_hi, hs_af)


def _small_body(xi_ref, wei_ref, bei_ref,
                hdhi_ref, hsrb_ref, hdrb_ref, hdaf_ref, cnt_ref,
                agghi_ref, aggaf_ref,
                w1hi_ref, b1hi_ref, w2hi_ref, b2hi_ref,
                w2rb_ref, b2rb_ref, w2af_ref, b2af_ref,
                wp1a_ref, wp1b_ref, bp1_ref,
                a_ref, bmat_ref):
    f32 = jnp.float32
    hp = jax.lax.Precision.HIGHEST
    ddi_hi = _dsi(jnp.sum(hdhi_ref[...], axis=0)[:NI])
    dsi_rb = _dsi(jnp.sum(hsrb_ref[...], axis=0)[:NI])
    ddi_rb = _dsi(jnp.sum(hdrb_ref[...], axis=0)[:NF])
    ddi_af = _dsi(jnp.sum(hdaf_ref[...], axis=0)[:NF])

    agg_hi = jnp.sum(agghi_ref[...], axis=0)
    agg_af = jnp.sum(aggaf_ref[...], axis=0)

    ii = jax.nn.relu(jnp.dot(xi_ref[...], wei_ref[...],
                             preferred_element_type=f32, precision=hp)
                     + bei_ref[...])
    M = jnp.sum(cnt_ref[...], axis=0) * dsi_rb[None, :]

    i1 = ddi_hi[:, None] * jnp.dot(agg_hi, w1hi_ref[...],
                                   preferred_element_type=f32, precision=hp) \
        + b1hi_ref[...]
    i2 = ddi_hi[:, None] * jnp.dot(agg_hi, w2hi_ref[...],
                                   preferred_element_type=f32, precision=hp) \
        + b2hi_ref[...]
    i1r = jax.nn.relu(i1)
    g2 = jnp.dot(M, i1r, preferred_element_type=f32, precision=hp)
    f2 = (ddi_rb[:, None] * jnp.dot(g2, w2rb_ref[...],
                                    preferred_element_type=f32, precision=hp)
          + b2rb_ref[...]
          + ddi_af[:, None] * jnp.dot(agg_af, w2af_ref[...],
                                      preferred_element_type=f32, precision=hp)
          + b2af_ref[...])

    a_ref[...] = jnp.dot(i2, wp1a_ref[...], preferred_element_type=f32,
                         precision=hp)
    bmat_ref[...] = jnp.dot(f2, wp1b_ref[...], preferred_element_type=f32,
                            precision=hp) + bp1_ref[...]


def _small_stage(x_intent, W_enc_i, b_enc_i, hd_hi, hs_rb, hd_rb, hd_af, cnt,
                 agg_hi_p, agg_af_p, W1_hi, b1_hi, W2_hi, b2_hi,
                 W2_rb, b2_rb, W2_af, b2_af, Wp1, bp1):
    return pl.pallas_call(
        _small_body,
        out_shape=[
            jax.ShapeDtypeStruct((NI, H), jnp.float32),
            jax.ShapeDtypeStruct((NF, H), jnp.float32),
        ],
        interpret=_INTERPRET,
    )(x_intent, W_enc_i, b_enc_i.reshape(1, H),
      hd_hi, hs_rb, hd_rb, hd_af, cnt,
      agg_hi_p, agg_af_p,
      W1_hi, b1_hi.reshape(1, H), W2_hi, b2_hi.reshape(1, H),
      W2_rb, b2_rb.reshape(1, H), W2_af, b2_af.reshape(1, H),
      Wp1[:H], Wp1[H:], bp1.reshape(1, H))


def _pair_body(a_ref, b_ref, wp2_ref, bp2_ref, wp3_ref, bp3_ref, out_ref, *, tr):
    f32 = jnp.float32
    hp = jax.lax.Precision.HIGHEST
    h1 = jax.nn.relu(a_ref[...][:, None, :] + b_ref[...][None, :, :])
    h1 = h1.reshape(tr * NF, H)
    h2 = jax.nn.relu(jnp.dot(h1, wp2_ref[...], preferred_element_type=f32,
                             precision=hp) + bp2_ref[...])
    logit = jnp.dot(h2, wp3_ref[...], preferred_element_type=f32,
                    precision=hp) + bp3_ref[...]
    out_ref[...] = jax.nn.sigmoid(logit.reshape(tr, NF))


def _pairwise(A, B, Wp2, bp2, Wp3, bp3):
    TR = 200
    grid = (NI // TR,)
    return pl.pallas_call(
        functools.partial(_pair_body, tr=TR),
        grid=grid,
        in_specs=[
            pl.BlockSpec((TR, H), lambda i: (i, 0)),
            pl.BlockSpec((NF, H), lambda i: (0, 0)),
            pl.BlockSpec((H, H // 2), lambda i: (0, 0)),
            pl.BlockSpec((1, H // 2), lambda i: (0, 0)),
            pl.BlockSpec((H // 2, 1), lambda i: (0, 0)),
            pl.BlockSpec((1, 1), lambda i: (0, 0)),
        ],
        out_specs=pl.BlockSpec((TR, NF), lambda i: (i, 0)),
        out_shape=jax.ShapeDtypeStruct((NI, NF), jnp.float32),
        interpret=_INTERPRET,
    )(A, B, Wp2, bp2.reshape(1, H // 2), Wp3, bp3.reshape(1, 1))


def kernel(x_dealer, x_intent, x_fix, edge_has_intent, edge_resolved_by,
           edge_applies_fix, W_enc_d, b_enc_d, W_enc_i, b_enc_i, W_enc_f,
           b_enc_f, W1_hi, b1_hi, W1_rb, b1_rb, W1_af, b1_af, W2_hi, b2_hi,
           W2_rb, b2_rb, W2_af, b2_af, Wp1, bp1, Wp2, bp2, Wp3, bp3):
    ehi, erb, eaf = edge_has_intent, edge_resolved_by, edge_applies_fix

    flat_rb = erb[1] * NI + erb[0]
    hs_hi, hd_hi, hs_af, hd_af, hs_rb, hd_rb, cnt = _sc_hist(
        _pad1(ehi[0], PAD_BIG, ND), _pad1(ehi[1], PAD_BIG, NI),
        _pad1(eaf[0], PAD_BIG, ND), _pad1(eaf[1], PAD_BIG, NF),
        _pad1(erb[0], PAD_RB, NI), _pad1(erb[1], PAD_RB, NF),
        _pad1(flat_rb, PAD_RB, NF * NI + NI))
    cnt = cnt[:, :NF * NI].reshape(32, NF, NI)

    d_hi, d_af = _encode_dealers(x_dealer, W_enc_d, b_enc_d, hs_hi, hs_af)

    agg_hi_p, agg_af_p = _sc_agg(
        d_hi.reshape(ND * NG, GF), d_af.reshape(ND * NG, GF),
        _pad1(ehi[0], PAD_Q, 0), _pad1(ehi[1], PAD_Q, THI_ROWS - 1),
        _pad1(eaf[0], PAD_Q, 0), _pad1(eaf[1], PAD_Q, TAF_ROWS - 1))

    A, B = _small_stage(x_intent, W_enc_i, b_enc_i, hd_hi, hs_rb, hd_rb, hd_af,
                        cnt, agg_hi_p, agg_af_p, W1_hi, b1_hi, W2_hi, b2_hi,
                        W2_rb, b2_rb, W2_af, b2_af, Wp1, bp1)
    return _pairwise(A, B, Wp2, bp2, Wp3, bp3)
